# baseline (speedup 1.0000x reference)
import functools

import jax
import jax.numpy as jnp
from jax import lax
from jax.experimental import pallas as pl
from jax.experimental.pallas import tpu as pltpu
from jax.experimental.pallas import tpu_sc as plsc

B = 4096
S = 200
D = 64
V = 1000
SCALE = 8.0

NC = 2
NS = 16
NW = NC * NS
B_PER_W = B // NW
RCHUNK = 4

_mesh = plsc.VectorSubcoreMesh(core_axis_name="c", subcore_axis_name="s")


@functools.partial(
    pl.kernel,
    out_type=jax.ShapeDtypeStruct((B, S, D), jnp.float32),
    mesh=_mesh,
    compiler_params=pltpu.CompilerParams(needs_layout_passes=False,
                                         use_tc_tiling_on_sc=False),
    scratch_types=[
        pltpu.VMEM((V * D,), jnp.float32),
        pltpu.VMEM((S * D,), jnp.float32),
        pltpu.VMEM((B_PER_W * D,), jnp.float32),
        pltpu.VMEM((RCHUNK * S + 16,), jnp.int32),
        pltpu.VMEM((B_PER_W,), jnp.int32),
        pltpu.VMEM((B_PER_W,), jnp.float32),
        pltpu.VMEM((D,), jnp.float32),
        pltpu.VMEM((D,), jnp.float32),
        pltpu.VMEM((S, D), jnp.float32),
        pltpu.VMEM((S, D), jnp.float32),
        pltpu.SemaphoreType.DMA,
        pltpu.SemaphoreType.DMA,
    ],
)
def _sc_embed(smiles_hbm, ads_hbm, chemo_hbm, table_hbm, ads_table_hbm,
              pos_hbm, w_hbm, cb_hbm, out_hbm,
              t_v, p_v, r_v, idx_s, adsid_v, chemo_v, w_v, cb_v,
              out0_v, out1_v, sem_o0, sem_o1):
    wid = lax.axis_index("s") * NC + lax.axis_index("c")
    base = wid * B_PER_W

    iota = lax.iota(jnp.int32, 16)
    col = [iota + 16 * j for j in range(4)]
    zero = jnp.zeros((16,), jnp.float32)

    pltpu.sync_copy(ads_table_hbm, t_v)
    pltpu.sync_copy(pos_hbm, p_v)
    pltpu.sync_copy(w_hbm, w_v)
    pltpu.sync_copy(cb_hbm, cb_v)
    pltpu.sync_copy(ads_hbm.at[pl.ds(base, B_PER_W)], adsid_v)
    pltpu.sync_copy(chemo_hbm.at[pl.ds(base, B_PER_W)], chemo_v)

    @plsc.parallel_loop(0, B_PER_W)
    def _r_loop(b):
        bv = jnp.full((16,), b, jnp.int32)
        chv8 = plsc.load_gather(chemo_v, [bv]) * SCALE
        aid64 = lax.shift_left(plsc.load_gather(adsid_v, [bv]), 6)
        for j in range(4):
            a = plsc.load_gather(t_v, [aid64 + col[j]])
            sl = pl.ds(16 * j, 16)
            r_v[pl.ds(b * D + 16 * j, 16)] = (
                a * SCALE + chv8 * w_v[sl] + cb_v[sl] * SCALE)

    pltpu.sync_copy(table_hbm, t_v)

    @plsc.parallel_loop(0, V * D, step=128)
    def _scale(i):
        for j in range(8):
            sl = pl.ds(i + 16 * j, 16)
            t_v[sl] = t_v[sl] * SCALE

    out_bufs = (out0_v, out1_v)
    out_sems = (sem_o0, sem_o1)

    @pl.loop(0, B_PER_W // RCHUNK)
    def _chunk(c):
        pltpu.sync_copy(
            smiles_hbm.at[pl.ds((base + c * RCHUNK) * S, RCHUNK * S)],
            idx_s.at[pl.ds(0, RCHUNK * S)])

        @pl.loop(0, RCHUNK // 2)
        def _pair(pr):
            for slot in range(2):
                r = pr * 2 + slot
                b = c * RCHUNK + r
                obuf = out_bufs[slot]
                osem = out_sems[slot]

                @pl.when(jnp.logical_or(c > 0, pr > 0))
                def _():
                    pltpu.make_async_copy(
                        obuf, out_hbm.at[0], osem).wait()

                rrow = [r_v[pl.ds(b * D + 16 * j, 16)] for j in range(4)]
                robase = r * S

                @plsc.parallel_loop(0, S, step=8)
                def _toks(s0):
                    t16 = idx_s[pl.ds(robase + s0, 16)]
                    for u in range(8):
                        s = s0 + u
                        tok64 = t16[u] * D
                        t64v = jnp.full((16,), tok64, jnp.int32)
                        m = t64v != 0
                        tidx = t64v + iota
                        po = s * D
                        for j in range(4):
                            g = plsc.load_gather(
                                t_v.at[pl.ds(16 * j, V * D - 16 * j)], [tidx])
                            pj = p_v[pl.ds(po + 16 * j, 16)]
                            val = jnp.where(m, g + (pj + rrow[j]), zero)
                            obuf[s, pl.ds(16 * j, 16)] = val

                pltpu.async_copy(obuf, out_hbm.at[base + b], osem)

    pltpu.make_async_copy(out0_v, out_hbm.at[0], sem_o0).wait()
    pltpu.make_async_copy(out1_v, out_hbm.at[0], sem_o1).wait()


def kernel(smiles, adsorbent, chemometrics, smile_table, ads_table, pos_table,
           chemo_W, chemo_b):
    return _sc_embed(smiles.reshape(B * S), adsorbent, chemometrics,
                     smile_table.reshape(V * D), ads_table.reshape(V * D),
                     pos_table.reshape(S * D), chemo_W.reshape(D), chemo_b)


# speedup vs baseline: 1.3438x; 1.3438x over previous
import functools

import jax
import jax.numpy as jnp
from jax import lax
from jax.experimental import pallas as pl
from jax.experimental.pallas import tpu as pltpu
from jax.experimental.pallas import tpu_sc as plsc

B = 4096
S = 200
D = 64
V = 1000
SCALE = 8.0
H0 = 104
H1 = 96

NC = 2
NS = 16
NW = NC * NS
B_PER_W = B // NW
RCHUNK = 4

_mesh = plsc.VectorSubcoreMesh(core_axis_name="c", subcore_axis_name="s")


@functools.partial(
    pl.kernel,
    out_type=jax.ShapeDtypeStruct((B, S, D), jnp.float32),
    mesh=_mesh,
    compiler_params=pltpu.CompilerParams(needs_layout_passes=False),
    scratch_types=[
        pltpu.VMEM((V * D,), jnp.float32),
        pltpu.VMEM((S * D,), jnp.float32),
        pltpu.VMEM((B_PER_W * D,), jnp.float32),
        pltpu.VMEM((RCHUNK * S + 16,), jnp.int32),
        pltpu.VMEM((B_PER_W,), jnp.int32),
        pltpu.VMEM((B_PER_W,), jnp.float32),
        pltpu.VMEM((D,), jnp.float32),
        pltpu.VMEM((D,), jnp.float32),
        pltpu.VMEM((H0, D), jnp.float32),
        pltpu.VMEM((H1, D), jnp.float32),
        pltpu.SemaphoreType.DMA,
        pltpu.SemaphoreType.DMA,
    ],
)
def _sc_embed(smiles_hbm, ads_hbm, chemo_hbm, table_hbm, ads_table_hbm,
              pos_hbm, w_hbm, cb_hbm, out_hbm,
              t_v, p_v, r_v, idx_s, adsid_v, chemo_v, w_v, cb_v,
              ob0, ob1, sm0, sm1):
    wid = lax.axis_index("s") * NC + lax.axis_index("c")
    base = wid * B_PER_W

    iota = lax.iota(jnp.int32, 16)
    col = [iota + 16 * j for j in range(4)]
    zero = jnp.zeros((16,), jnp.float32)

    pltpu.sync_copy(ads_table_hbm, t_v)
    pltpu.sync_copy(pos_hbm, p_v)
    pltpu.sync_copy(w_hbm, w_v)
    pltpu.sync_copy(cb_hbm, cb_v)
    pltpu.sync_copy(ads_hbm.at[pl.ds(base, B_PER_W)], adsid_v)
    pltpu.sync_copy(chemo_hbm.at[pl.ds(base, B_PER_W)], chemo_v)

    @plsc.parallel_loop(0, B_PER_W)
    def _r_loop(b):
        bv = jnp.full((16,), b, jnp.int32)
        chv8 = plsc.load_gather(chemo_v, [bv]) * SCALE
        aid64 = lax.shift_left(plsc.load_gather(adsid_v, [bv]), 6)
        for j in range(4):
            a = plsc.load_gather(t_v, [aid64 + col[j]])
            sl = pl.ds(16 * j, 16)
            r_v[pl.ds(b * D + 16 * j, 16)] = (
                a * SCALE + chv8 * w_v[sl] + cb_v[sl] * SCALE)

    pltpu.sync_copy(table_hbm, t_v)

    @plsc.parallel_loop(0, V * D, step=128)
    def _scale(i):
        for j in range(8):
            sl = pl.ds(i + 16 * j, 16)
            t_v[sl] = t_v[sl] * SCALE

    bufs = (ob0, ob1)
    sems = (sm0, sm1)
    hstart = (0, H0)
    hlen = (H0, H1)

    @pl.loop(0, B_PER_W // RCHUNK)
    def _chunk(c):
        pltpu.sync_copy(
            smiles_hbm.at[pl.ds((base + c * RCHUNK) * S, RCHUNK * S)],
            idx_s.at[pl.ds(0, RCHUNK * S)])

        @pl.loop(0, RCHUNK)
        def _row(r):
            b = c * RCHUNK + r
            rr = [r_v[pl.ds(b * D + 16 * j, 16)] for j in range(4)]
            robase = r * S

            for h in range(2):
                obuf = bufs[h]
                osem = sems[h]

                @pl.when(jnp.logical_or(c > 0, r > 0))
                def _():
                    pltpu.make_async_copy(
                        obuf, out_hbm.at[0, pl.ds(hstart[h], hlen[h])],
                        osem).wait()

                @plsc.parallel_loop(0, hlen[h], step=8)
                def _toks(s0):
                    t16 = idx_s[pl.ds(robase + hstart[h] + s0, 16)]
                    for u in range(8):
                        sl_ = s0 + u
                        s = hstart[h] + sl_
                        tok64 = t16[u] * D
                        t64v = jnp.full((16,), tok64, jnp.int32)
                        m = t64v != 0
                        tidx = t64v + iota
                        for j in range(4):
                            g = plsc.load_gather(
                                t_v.at[pl.ds(16 * j, V * D - 16 * j)], [tidx])
                            pj = p_v[pl.ds(s * D + 16 * j, 16)]
                            val = jnp.where(m, g + (pj + rr[j]), zero)
                            obuf[sl_, pl.ds(16 * j, 16)] = val

                pltpu.async_copy(
                    obuf, out_hbm.at[base + b, pl.ds(hstart[h], hlen[h])],
                    osem)

    for h in range(2):
        pltpu.make_async_copy(bufs[h],
                              out_hbm.at[0, pl.ds(hstart[h], hlen[h])],
                              sems[h]).wait()


def kernel(smiles, adsorbent, chemometrics, smile_table, ads_table, pos_table,
           chemo_W, chemo_b):
    return _sc_embed(smiles.reshape(B * S), adsorbent, chemometrics,
                     smile_table.reshape(V * D), ads_table.reshape(V * D),
                     pos_table.reshape(S * D), chemo_W.reshape(D), chemo_b)
